# baseline (speedup 1.0000x reference)
import functools

import jax
import jax.numpy as jnp
from jax import lax
from jax.experimental import pallas as pl
from jax.experimental.pallas import tpu as pltpu
from jax.experimental.pallas import tpu_sc as plsc

_N, _E, _F, _H, _G, _C = 10000, 320000, 128, 64, 64, 10
_NC, _NS = 2, 16
_NW = _NC * _NS
_EPW = _E // _NW
_K = 80
_NCHUNK = _EPW // _K
_NPAD = 10240
_RPT = _NPAD // _NS
_ZR = 128


@functools.partial(
    pl.kernel,
    mesh=plsc.VectorSubcoreMesh(core_axis_name="c", subcore_axis_name="s"),
    out_type=jax.ShapeDtypeStruct((_NC, _NPAD, _H), jnp.float32),
    compiler_params=pltpu.CompilerParams(use_tc_tiling_on_sc=False),
    scratch_types=[
        pltpu.VMEM((_NCHUNK, _K), jnp.int32),
        pltpu.VMEM((_NCHUNK, _K), jnp.int32),
        pltpu.VMEM((_K, _H), jnp.float32),
        pltpu.VMEM((_K, _H), jnp.float32),
        pltpu.VMEM((_ZR, _H), jnp.float32),
        pltpu.VMEM_SHARED((_NPAD, _H), jnp.float32),
        pltpu.VMEM_SHARED((_NPAD, _H), jnp.float32),
        pltpu.SemaphoreType.DMA,
        pltpu.SemaphoreType.DMA,
    ],
)
def _edge_scatter(y_hbm, src_hbm, dst_hbm, out_hbm,
                  src_v, dst_v, rows0, rows1, zbuf, acc, ytab, sem0, sem1):
    c = lax.axis_index("c")
    s = lax.axis_index("s")
    wid = c * _NS + s

    zero16 = jnp.zeros((16,), jnp.float32)

    def zrow(r, carry):
        for c4 in range(_H // 16):
            zbuf[r, pl.ds(c4 * 16, 16)] = zero16
        return carry

    pltpu.sync_copy(y_hbm.at[pl.ds(s * _RPT, _RPT)],
                    ytab.at[pl.ds(s * _RPT, _RPT)])

    lax.fori_loop(0, _ZR, zrow, 0)
    for b in range(_RPT // _ZR):
        pltpu.sync_copy(zbuf, acc.at[pl.ds(s * _RPT + b * _ZR, _ZR)])
    plsc.subcore_barrier()

    pltpu.sync_copy(src_hbm.at[wid], src_v)
    pltpu.sync_copy(dst_hbm.at[wid], dst_v)

    def gather_start(j, tab, buf, sem):
        pltpu.async_copy(tab.at[src_v.at[j]], buf, sem)

    def gather_wait(j, tab, buf, sem):
        pltpu.make_async_copy(tab.at[src_v.at[j]], buf, sem).wait()

    gather_start(0, y_hbm, rows0, sem0)

    def chunk2(g, carry):
        j0 = g * 2
        gather_start(j0 + 1, ytab, rows1, sem1)
        gather_wait(j0, y_hbm, rows0, sem0)
        pltpu.sync_copy(rows0, acc.at[dst_v.at[j0]], add=True)
        gather_start(j0 + 2, y_hbm, rows0, sem0)
        gather_wait(j0 + 1, ytab, rows1, sem1)
        pltpu.sync_copy(rows1, acc.at[dst_v.at[j0 + 1]], add=True)
        return carry

    lax.fori_loop(0, (_NCHUNK - 1) // 2, chunk2, 0)
    gather_wait(_NCHUNK - 1, y_hbm, rows0, sem0)
    pltpu.sync_copy(rows0, acc.at[dst_v.at[_NCHUNK - 1]], add=True)
    plsc.subcore_barrier()

    pltpu.sync_copy(acc.at[pl.ds(s * _RPT, _RPT)],
                    out_hbm.at[c, pl.ds(s * _RPT, _RPT)])


def _bn_relu(h, g, t):
    m = jnp.mean(h, axis=0, keepdims=True)
    v = jnp.mean((h - m) * (h - m), axis=0, keepdims=True)
    return jnp.maximum((h - m) / jnp.sqrt(v + 1e-5) * g + t, 0.0)


def _mm_body(x_ref, w_ref, o_ref):
    o_ref[...] = jnp.dot(x_ref[...], w_ref[...],
                         preferred_element_type=jnp.float32)


def _mlp_body(y_ref, p_ref, ba_ref, ga_ref, ta_ref,
              Wb_ref, bb_ref, gb_ref, tb_ref, Wnext_ref, o_ref):
    pre = y_ref[...] + p_ref[0, :_N] + p_ref[1, :_N] + ba_ref[...]
    h = _bn_relu(pre, ga_ref[...], ta_ref[...])
    h = _bn_relu(jnp.dot(h, Wb_ref[...], preferred_element_type=jnp.float32)
                 + bb_ref[...], gb_ref[...], tb_ref[...])
    o_ref[...] = jnp.dot(h, Wnext_ref[...], preferred_element_type=jnp.float32)


def _tail_body(y_ref, q_ref, batch_ref, ba_ref, ga_ref, ta_ref,
               Wb_ref, bb_ref, gb_ref, tb_ref,
               Wc1_ref, bc1_ref, gc1_ref, tc1_ref, Wc2_ref, bc2_ref, o_ref):
    pre = y_ref[...] + q_ref[0, :_N] + q_ref[1, :_N] + ba_ref[...]
    h = _bn_relu(pre, ga_ref[...], ta_ref[...])
    h = _bn_relu(jnp.dot(h, Wb_ref[...], preferred_element_type=jnp.float32)
                 + bb_ref[...], gb_ref[...], tb_ref[...])
    gids = lax.broadcasted_iota(jnp.int32, (_G, _N), 0)
    oh = (gids == batch_ref[...]).astype(jnp.float32)
    sums = jnp.dot(oh, h, preferred_element_type=jnp.float32)
    cnt = jnp.sum(oh, axis=1, keepdims=True)
    hm = sums / jnp.maximum(cnt, 1.0)
    z = _bn_relu(jnp.dot(hm, Wc1_ref[...], preferred_element_type=jnp.float32)
                 + bc1_ref[...], gc1_ref[...], tc1_ref[...])
    z = jnp.dot(z, Wc2_ref[...], preferred_element_type=jnp.float32) + bc2_ref[...]
    zmax = jnp.max(z, axis=1, keepdims=True)
    zs = z - zmax
    o_ref[...] = zs - jnp.log(jnp.sum(jnp.exp(zs), axis=1, keepdims=True))


_mm = pl.pallas_call(
    _mm_body, out_shape=jax.ShapeDtypeStruct((_N, _H), jnp.float32))

_mlp = pl.pallas_call(
    _mlp_body, out_shape=jax.ShapeDtypeStruct((_N, _H), jnp.float32))

_tail = pl.pallas_call(
    _tail_body, out_shape=jax.ShapeDtypeStruct((_G, _C), jnp.float32))


def kernel(x, edge_index, batch, W1a, b1a, g1a, t1a, W1b, b1b, g1b, t1b,
           W2a, b2a, g2a, t2a, W2b, b2b, g2b, t2b,
           Wc1, bc1, gc1, tc1, Wc2, bc2):
    r = lambda v: v.reshape(1, -1)

    src3 = edge_index[0].reshape(_NW, _NCHUNK, _K)
    dst3 = edge_index[1].reshape(_NW, _NCHUNK, _K)

    pad = lambda v: jnp.pad(v, ((0, _NPAD - _N), (0, 0)))

    es = jax.jit(_edge_scatter)
    y1 = _mm(x, W1a)
    p = es(pad(y1), src3, dst3)
    y2 = _mlp(y1, p, r(b1a), r(g1a), r(t1a), W1b, r(b1b), r(g1b), r(t1b), W2a)
    q = es(pad(y2), src3, dst3)
    return _tail(y2, q, batch.reshape(1, _N), r(b2a), r(g2a), r(t2a),
                 W2b, r(b2b), r(g2b), r(t2b),
                 Wc1, r(bc1), r(gc1), r(tc1), Wc2, r(bc2))


# speedup vs baseline: 1.2473x; 1.2473x over previous
import functools

import jax
import jax.numpy as jnp
from jax import lax
from jax.experimental import pallas as pl
from jax.experimental.pallas import tpu as pltpu
from jax.experimental.pallas import tpu_sc as plsc

_N, _E, _F, _H, _G, _C = 10000, 320000, 128, 64, 64, 10
_NC, _NS = 2, 16
_NW = _NC * _NS
_EPW = _E // _NW
_K = 80
_NCHUNK = _EPW // _K
_NPAD = 10240
_RPT = _NPAD // _NS
_ZR = 128


@functools.partial(
    pl.kernel,
    mesh=plsc.VectorSubcoreMesh(core_axis_name="c", subcore_axis_name="s"),
    out_type=jax.ShapeDtypeStruct((_NC, _NPAD, _H), jnp.float32),
    compiler_params=pltpu.CompilerParams(use_tc_tiling_on_sc=False),
    scratch_types=[
        pltpu.VMEM((_NCHUNK, _K), jnp.int32),
        pltpu.VMEM((_NCHUNK, _K), jnp.int32),
        pltpu.VMEM((_K, _H), jnp.float32),
        pltpu.VMEM((_K, _H), jnp.float32),
        pltpu.VMEM((_K, _H), jnp.float32),
        pltpu.VMEM((_K, _H), jnp.float32),
        pltpu.VMEM((_ZR, _H), jnp.float32),
        pltpu.VMEM_SHARED((_NPAD, _H), jnp.float32),
        pltpu.VMEM_SHARED((_NPAD, _H), jnp.float32),
        pltpu.SemaphoreType.DMA, pltpu.SemaphoreType.DMA,
        pltpu.SemaphoreType.DMA, pltpu.SemaphoreType.DMA,
        pltpu.SemaphoreType.DMA, pltpu.SemaphoreType.DMA,
        pltpu.SemaphoreType.DMA, pltpu.SemaphoreType.DMA,
    ],
)
def _edge_scatter(y_hbm, src_hbm, dst_hbm, out_hbm,
                  src_v, dst_v, rows0, rows1, rows2, rows3,
                  zbuf, acc, ytab,
                  gsem0, gsem1, gsem2, gsem3,
                  ssem0, ssem1, ssem2, ssem3):
    c = lax.axis_index("c")
    s = lax.axis_index("s")
    wid = c * _NS + s

    zero16 = jnp.zeros((16,), jnp.float32)

    def zrow(r, carry):
        for c4 in range(_H // 16):
            zbuf[r, pl.ds(c4 * 16, 16)] = zero16
        return carry

    pltpu.sync_copy(y_hbm.at[pl.ds(s * _RPT, _RPT)],
                    ytab.at[pl.ds(s * _RPT, _RPT)])

    lax.fori_loop(0, _ZR, zrow, 0)
    for b in range(_RPT // _ZR):
        pltpu.sync_copy(zbuf, acc.at[pl.ds(s * _RPT + b * _ZR, _ZR)])
    plsc.subcore_barrier()

    pltpu.sync_copy(src_hbm.at[wid], src_v)
    pltpu.sync_copy(dst_hbm.at[wid], dst_v)

    rows = (rows0, rows1, rows2, rows3)
    gsems = (gsem0, gsem1, gsem2, gsem3)
    ssems = (ssem0, ssem1, ssem2, ssem3)

    def gather_start(j, b):
        pltpu.async_copy(ytab.at[src_v.at[j]], rows[b], gsems[b])

    def gather_wait(j, b):
        pltpu.make_async_copy(ytab.at[src_v.at[j]], rows[b], gsems[b]).wait()

    def scatter_start(j, b):
        pltpu.async_copy(rows[b], acc.at[dst_v.at[j]], ssems[b], add=True)

    def scatter_wait(j, b):
        pltpu.make_async_copy(rows[b], acc.at[dst_v.at[j]], ssems[b]).wait()

    def do_chunk(j, b, first, last):
        b2 = (b + 2) % 4
        if not first:
            scatter_wait(j - 2, b2)
        if not last:
            gather_start(j + 2, b2)
        gather_wait(j, b)
        scatter_start(j, b)

    for j in range(2):
        gather_start(j, j)
    for j in range(4):
        do_chunk(j, j, first=(j < 2), last=False)

    def body4(g, carry):
        j0 = g * 4
        for b in range(4):
            do_chunk(j0 + b, b, first=False, last=False)
        return carry

    lax.fori_loop(1, (_NCHUNK - 5) // 4, body4, 0)
    for j in range(_NCHUNK - 5, _NCHUNK):
        do_chunk(j, j % 4, first=False, last=(j + 2 >= _NCHUNK))
    for j in range(_NCHUNK - 2, _NCHUNK):
        scatter_wait(j, j % 4)
    plsc.subcore_barrier()

    pltpu.sync_copy(acc.at[pl.ds(s * _RPT, _RPT)],
                    out_hbm.at[c, pl.ds(s * _RPT, _RPT)])


def _bn_relu(h, g, t):
    m = jnp.mean(h, axis=0, keepdims=True)
    v = jnp.mean((h - m) * (h - m), axis=0, keepdims=True)
    return jnp.maximum((h - m) / jnp.sqrt(v + 1e-5) * g + t, 0.0)


def _mm_body(x_ref, w_ref, o_ref):
    o_ref[...] = jnp.dot(x_ref[...], w_ref[...],
                         preferred_element_type=jnp.float32)


def _mlp_body(y_ref, p_ref, ba_ref, ga_ref, ta_ref,
              Wb_ref, bb_ref, gb_ref, tb_ref, Wnext_ref, o_ref):
    pre = y_ref[...] + p_ref[0, :_N] + p_ref[1, :_N] + ba_ref[...]
    h = _bn_relu(pre, ga_ref[...], ta_ref[...])
    h = _bn_relu(jnp.dot(h, Wb_ref[...], preferred_element_type=jnp.float32)
                 + bb_ref[...], gb_ref[...], tb_ref[...])
    o_ref[...] = jnp.dot(h, Wnext_ref[...], preferred_element_type=jnp.float32)


def _tail_body(y_ref, q_ref, batch_ref, ba_ref, ga_ref, ta_ref,
               Wb_ref, bb_ref, gb_ref, tb_ref,
               Wc1_ref, bc1_ref, gc1_ref, tc1_ref, Wc2_ref, bc2_ref, o_ref):
    pre = y_ref[...] + q_ref[0, :_N] + q_ref[1, :_N] + ba_ref[...]
    h = _bn_relu(pre, ga_ref[...], ta_ref[...])
    h = _bn_relu(jnp.dot(h, Wb_ref[...], preferred_element_type=jnp.float32)
                 + bb_ref[...], gb_ref[...], tb_ref[...])
    gids = lax.broadcasted_iota(jnp.int32, (_G, _N), 0)
    oh = (gids == batch_ref[...]).astype(jnp.float32)
    sums = jnp.dot(oh, h, preferred_element_type=jnp.float32)
    cnt = jnp.sum(oh, axis=1, keepdims=True)
    hm = sums / jnp.maximum(cnt, 1.0)
    z = _bn_relu(jnp.dot(hm, Wc1_ref[...], preferred_element_type=jnp.float32)
                 + bc1_ref[...], gc1_ref[...], tc1_ref[...])
    z = jnp.dot(z, Wc2_ref[...], preferred_element_type=jnp.float32) + bc2_ref[...]
    zmax = jnp.max(z, axis=1, keepdims=True)
    zs = z - zmax
    o_ref[...] = zs - jnp.log(jnp.sum(jnp.exp(zs), axis=1, keepdims=True))


_mm = pl.pallas_call(
    _mm_body, out_shape=jax.ShapeDtypeStruct((_N, _H), jnp.float32))

_mlp = pl.pallas_call(
    _mlp_body, out_shape=jax.ShapeDtypeStruct((_N, _H), jnp.float32))

_tail = pl.pallas_call(
    _tail_body, out_shape=jax.ShapeDtypeStruct((_G, _C), jnp.float32))


def kernel(x, edge_index, batch, W1a, b1a, g1a, t1a, W1b, b1b, g1b, t1b,
           W2a, b2a, g2a, t2a, W2b, b2b, g2b, t2b,
           Wc1, bc1, gc1, tc1, Wc2, bc2):
    r = lambda v: v.reshape(1, -1)

    src3 = edge_index[0].reshape(_NW, _NCHUNK, _K)
    dst3 = edge_index[1].reshape(_NW, _NCHUNK, _K)

    pad = lambda v: jnp.pad(v, ((0, _NPAD - _N), (0, 0)))

    es = jax.jit(_edge_scatter)
    y1 = _mm(x, W1a)
    p = es(pad(y1), src3, dst3)
    y2 = _mlp(y1, p, r(b1a), r(g1a), r(t1a), W1b, r(b1b), r(g1b), r(t1b), W2a)
    q = es(pad(y2), src3, dst3)
    return _tail(y2, q, batch.reshape(1, _N), r(b2a), r(g2a), r(t2a),
                 W2b, r(b2b), r(g2b), r(t2b),
                 Wc1, r(bc1), r(gc1), r(tc1), Wc2, r(bc2))


# speedup vs baseline: 1.4755x; 1.1830x over previous
import functools

import jax
import jax.numpy as jnp
from jax import lax
from jax.experimental import pallas as pl
from jax.experimental.pallas import tpu as pltpu
from jax.experimental.pallas import tpu_sc as plsc

_N, _E, _F, _H, _G, _C = 10000, 320000, 128, 64, 64, 10
_NC, _NS = 2, 16
_NW = _NC * _NS
_EPW = _E // _NW
_K = 80
_NCHUNK = _EPW // _K
_NPAD = 10240
_RPT = _NPAD // _NS
_ZR = 128


@functools.partial(
    pl.kernel,
    mesh=plsc.VectorSubcoreMesh(core_axis_name="c", subcore_axis_name="s"),
    out_type=jax.ShapeDtypeStruct((_NC, _NPAD, _H), jnp.float32),
    compiler_params=pltpu.CompilerParams(use_tc_tiling_on_sc=False),
    scratch_types=[
        pltpu.VMEM((_NCHUNK, _K), jnp.int32),
        pltpu.VMEM((_NCHUNK, _K), jnp.int32),
        pltpu.VMEM((_K, _H), jnp.float32),
        pltpu.VMEM((_K, _H), jnp.float32),
        pltpu.VMEM((_K, _H), jnp.float32),
        pltpu.VMEM((_K, _H), jnp.float32),
        pltpu.VMEM((_ZR, _H), jnp.float32),
        pltpu.VMEM_SHARED((_NPAD, _H), jnp.float32),
        pltpu.VMEM_SHARED((_NPAD, _H), jnp.float32),
        pltpu.SemaphoreType.DMA, pltpu.SemaphoreType.DMA,
        pltpu.SemaphoreType.DMA, pltpu.SemaphoreType.DMA,
        pltpu.SemaphoreType.DMA, pltpu.SemaphoreType.DMA,
        pltpu.SemaphoreType.DMA, pltpu.SemaphoreType.DMA,
    ],
)
def _edge_scatter(y_hbm, src_hbm, dst_hbm, out_hbm,
                  src_v, dst_v, rows0, rows1, rows2, rows3,
                  zbuf, acc, ytab,
                  gsem0, gsem1, gsem2, gsem3,
                  ssem0, ssem1, ssem2, ssem3):
    c = lax.axis_index("c")
    s = lax.axis_index("s")
    wid = c * _NS + s

    zero16 = jnp.zeros((16,), jnp.float32)

    def zrow(r, carry):
        for c4 in range(_H // 16):
            zbuf[r, pl.ds(c4 * 16, 16)] = zero16
        return carry

    pltpu.sync_copy(y_hbm.at[pl.ds(s * _RPT, _RPT)],
                    ytab.at[pl.ds(s * _RPT, _RPT)])

    lax.fori_loop(0, _ZR, zrow, 0)
    for b in range(_RPT // _ZR):
        pltpu.sync_copy(zbuf, acc.at[pl.ds(s * _RPT + b * _ZR, _ZR)])
    plsc.subcore_barrier()

    pltpu.sync_copy(src_hbm.at[wid], src_v)
    pltpu.sync_copy(dst_hbm.at[wid], dst_v)

    rows = (rows0, rows1, rows2, rows3)
    gsems = (gsem0, gsem1, gsem2, gsem3)
    ssems = (ssem0, ssem1, ssem2, ssem3)

    def gather_start(j, b):
        pltpu.async_copy(ytab.at[src_v.at[j]], rows[b], gsems[b])

    def gather_wait(j, b):
        pltpu.make_async_copy(ytab.at[src_v.at[j]], rows[b], gsems[b]).wait()

    def scatter_start(j, b):
        pltpu.async_copy(rows[b], acc.at[dst_v.at[j]], ssems[b], add=True)

    def scatter_wait(j, b):
        pltpu.make_async_copy(rows[b], acc.at[dst_v.at[j]], ssems[b]).wait()

    def do_chunk(j, b, first, last):
        b2 = (b + 2) % 4
        if not first:
            scatter_wait(j - 2, b2)
        if not last:
            gather_start(j + 2, b2)
        gather_wait(j, b)
        scatter_start(j, b)

    for j in range(2):
        gather_start(j, j)
    for j in range(4):
        do_chunk(j, j, first=(j < 2), last=False)

    def body4(g, carry):
        j0 = g * 4
        for b in range(4):
            do_chunk(j0 + b, b, first=False, last=False)
        return carry

    lax.fori_loop(1, (_NCHUNK - 5) // 4, body4, 0)
    for j in range(_NCHUNK - 5, _NCHUNK):
        do_chunk(j, j % 4, first=False, last=(j + 2 >= _NCHUNK))
    for j in range(_NCHUNK - 2, _NCHUNK):
        scatter_wait(j, j % 4)
    plsc.subcore_barrier()

    pltpu.sync_copy(acc.at[pl.ds(s * _RPT, _RPT)],
                    out_hbm.at[c, pl.ds(s * _RPT, _RPT)])


_NP2 = _NPAD // 2
_NR2 = _N // 2


def _bn_relu2(hL, hR, g, t):
    sm = jnp.sum(hL, 0, keepdims=True) + jnp.sum(hR, 0, keepdims=True)
    m = sm / _N
    s2 = (jnp.sum(hL * hL, 0, keepdims=True)
          + jnp.sum(hR * hR, 0, keepdims=True))
    v = s2 / _N - m * m
    inv = g / jnp.sqrt(v + 1e-5)
    fL = jnp.maximum((hL - m) * inv + t, 0.0)
    fR = jnp.maximum((hR - m) * inv + t, 0.0)
    return fL, fR


def _mm_body(x_ref, w_ref, o_ref):
    yL = jnp.dot(x_ref[:, :_F], w_ref[...], preferred_element_type=jnp.float32)
    yR = jnp.dot(x_ref[:, _F:], w_ref[...], preferred_element_type=jnp.float32)
    o_ref[:_NR2] = jnp.concatenate([yL, yR], axis=1)
    o_ref[_NR2:] = jnp.zeros((_NP2 - _NR2, 2 * _H), jnp.float32)


def _mlp_core(y_ref, p_ref, ba, ga, ta, Wb_ref, bb, gb, tb):
    pre = y_ref[:_NR2] + p_ref[0, :_NR2] + p_ref[1, :_NR2]
    hL, hR = _bn_relu2(pre[:, :_H] + ba, pre[:, _H:] + ba, ga, ta)
    zL = jnp.dot(hL, Wb_ref[...], preferred_element_type=jnp.float32) + bb
    zR = jnp.dot(hR, Wb_ref[...], preferred_element_type=jnp.float32) + bb
    return _bn_relu2(zL, zR, gb, tb)


def _mlp_body(y_ref, p_ref, ba_ref, ga_ref, ta_ref,
              Wb_ref, bb_ref, gb_ref, tb_ref, Wnext_ref, o_ref):
    hL, hR = _mlp_core(y_ref, p_ref, ba_ref[...], ga_ref[...], ta_ref[...],
                       Wb_ref, bb_ref[...], gb_ref[...], tb_ref[...])
    yL = jnp.dot(hL, Wnext_ref[...], preferred_element_type=jnp.float32)
    yR = jnp.dot(hR, Wnext_ref[...], preferred_element_type=jnp.float32)
    o_ref[:_NR2] = jnp.concatenate([yL, yR], axis=1)
    o_ref[_NR2:] = jnp.zeros((_NP2 - _NR2, 2 * _H), jnp.float32)


def _tail_body(y_ref, q_ref, bE_ref, bO_ref, ba_ref, ga_ref, ta_ref,
               Wb_ref, bb_ref, gb_ref, tb_ref,
               Wc1_ref, bc1_ref, gc1_ref, tc1_ref, Wc2_ref, bc2_ref, o_ref):
    hL, hR = _mlp_core(y_ref, q_ref, ba_ref[...], ga_ref[...], ta_ref[...],
                       Wb_ref, bb_ref[...], gb_ref[...], tb_ref[...])
    gids = lax.broadcasted_iota(jnp.int32, (_G, _NR2), 0)
    ohE = (gids == bE_ref[...]).astype(jnp.float32)
    ohO = (gids == bO_ref[...]).astype(jnp.float32)
    sums = (jnp.dot(ohE, hL, preferred_element_type=jnp.float32)
            + jnp.dot(ohO, hR, preferred_element_type=jnp.float32))
    cnt = (jnp.sum(ohE, axis=1, keepdims=True)
           + jnp.sum(ohO, axis=1, keepdims=True))
    hm = sums / jnp.maximum(cnt, 1.0)
    z = jnp.dot(hm, Wc1_ref[...], preferred_element_type=jnp.float32) + bc1_ref[...]
    m = jnp.mean(z, axis=0, keepdims=True)
    v = jnp.mean((z - m) * (z - m), axis=0, keepdims=True)
    z = jnp.maximum((z - m) / jnp.sqrt(v + 1e-5) * gc1_ref[...] + tc1_ref[...], 0.0)
    z = jnp.dot(z, Wc2_ref[...], preferred_element_type=jnp.float32) + bc2_ref[...]
    zmax = jnp.max(z, axis=1, keepdims=True)
    zs = z - zmax
    o_ref[...] = zs - jnp.log(jnp.sum(jnp.exp(zs), axis=1, keepdims=True))


_mm = pl.pallas_call(
    _mm_body, out_shape=jax.ShapeDtypeStruct((_NP2, 2 * _H), jnp.float32))

_mlp = pl.pallas_call(
    _mlp_body, out_shape=jax.ShapeDtypeStruct((_NP2, 2 * _H), jnp.float32))

_tail = pl.pallas_call(
    _tail_body, out_shape=jax.ShapeDtypeStruct((_G, _C), jnp.float32))


def kernel(x, edge_index, batch, W1a, b1a, g1a, t1a, W1b, b1b, g1b, t1b,
           W2a, b2a, g2a, t2a, W2b, b2b, g2b, t2b,
           Wc1, bc1, gc1, tc1, Wc2, bc2):
    r = lambda v: v.reshape(1, -1)

    src3 = edge_index[0].reshape(_NW, _NCHUNK, _K)
    dst3 = edge_index[1].reshape(_NW, _NCHUNK, _K)
    xp = x.reshape(_NR2, 2 * _F)
    bpair = batch.reshape(_NR2, 2)
    bE = bpair[:, 0].reshape(1, _NR2)
    bO = bpair[:, 1].reshape(1, _NR2)

    tosc = lambda v: v.reshape(_NPAD, _H)
    topair = lambda v: v.reshape(_NC, _NP2, 2 * _H)

    es = jax.jit(_edge_scatter)
    y1 = _mm(xp, W1a)
    p = es(tosc(y1), src3, dst3)
    y2 = _mlp(y1, topair(p), r(b1a), r(g1a), r(t1a),
              W1b, r(b1b), r(g1b), r(t1b), W2a)
    q = es(tosc(y2), src3, dst3)
    return _tail(y2, topair(q), bE, bO, r(b2a), r(g2a), r(t2a),
                 W2b, r(b2b), r(g2b), r(t2b),
                 Wc1, r(bc1), r(gc1), r(tc1), Wc2, r(bc2))


# speedup vs baseline: 1.5458x; 1.0476x over previous
import functools

import jax
import jax.numpy as jnp
from jax import lax
from jax.experimental import pallas as pl
from jax.experimental.pallas import tpu as pltpu
from jax.experimental.pallas import tpu_sc as plsc

_N, _E, _F, _H, _G, _C = 10000, 320000, 128, 64, 64, 10
_NC, _NS = 2, 16
_NW = _NC * _NS
_EPW = _E // _NW
_K = 80
_NCHUNK = _EPW // _K
_NPAD = 10240
_RPT = _NPAD // _NS
_ZR = 128


@functools.partial(
    pl.kernel,
    mesh=plsc.VectorSubcoreMesh(core_axis_name="c", subcore_axis_name="s"),
    out_type=jax.ShapeDtypeStruct((_NC, _NPAD, _H), jnp.float32),
    compiler_params=pltpu.CompilerParams(use_tc_tiling_on_sc=False),
    scratch_types=[
        pltpu.VMEM((_NCHUNK, _K), jnp.int32),
        pltpu.VMEM((_NCHUNK, _K), jnp.int32),
        pltpu.VMEM((_K, _H), jnp.float32),
        pltpu.VMEM((_K, _H), jnp.float32),
        pltpu.VMEM((_K, _H), jnp.float32),
        pltpu.VMEM((_K, _H), jnp.float32),
        pltpu.VMEM((_ZR, _H), jnp.float32),
        pltpu.VMEM_SHARED((_NPAD, _H), jnp.float32),
        pltpu.VMEM_SHARED((_NPAD, _H), jnp.float32),
        pltpu.SemaphoreType.DMA, pltpu.SemaphoreType.DMA,
        pltpu.SemaphoreType.DMA, pltpu.SemaphoreType.DMA,
        pltpu.SemaphoreType.DMA, pltpu.SemaphoreType.DMA,
        pltpu.SemaphoreType.DMA, pltpu.SemaphoreType.DMA,
    ],
)
def _edge_scatter(y_hbm, ei_hbm, out_hbm,
                  src_v, dst_v, rows0, rows1, rows2, rows3,
                  zbuf, acc, ytab,
                  gsem0, gsem1, gsem2, gsem3,
                  ssem0, ssem1, ssem2, ssem3):
    c = lax.axis_index("c")
    s = lax.axis_index("s")
    wid = c * _NS + s

    zero16 = jnp.zeros((16,), jnp.float32)

    def zrow(r, carry):
        for c4 in range(_H // 16):
            zbuf[r, pl.ds(c4 * 16, 16)] = zero16
        return carry

    pltpu.sync_copy(y_hbm.at[pl.ds(s * _RPT, _RPT)],
                    ytab.at[pl.ds(s * _RPT, _RPT)])

    lax.fori_loop(0, _ZR, zrow, 0)
    for b in range(_RPT // _ZR):
        pltpu.sync_copy(zbuf, acc.at[pl.ds(s * _RPT + b * _ZR, _ZR)])
    plsc.subcore_barrier()

    pltpu.sync_copy(ei_hbm.at[0, wid], src_v)
    pltpu.sync_copy(ei_hbm.at[1, wid], dst_v)

    rows = (rows0, rows1, rows2, rows3)
    gsems = (gsem0, gsem1, gsem2, gsem3)
    ssems = (ssem0, ssem1, ssem2, ssem3)

    def gather_start(j, b):
        pltpu.async_copy(ytab.at[src_v.at[j]], rows[b], gsems[b])

    def gather_wait(j, b):
        pltpu.make_async_copy(ytab.at[src_v.at[j]], rows[b], gsems[b]).wait()

    def scatter_start(j, b):
        pltpu.async_copy(rows[b], acc.at[dst_v.at[j]], ssems[b], add=True)

    def scatter_wait(j, b):
        pltpu.make_async_copy(rows[b], acc.at[dst_v.at[j]], ssems[b]).wait()

    def do_chunk(j, b, first, last):
        b2 = (b + 2) % 4
        if not first:
            scatter_wait(j - 2, b2)
        if not last:
            gather_start(j + 2, b2)
        gather_wait(j, b)
        scatter_start(j, b)

    for j in range(2):
        gather_start(j, j)
    for j in range(4):
        do_chunk(j, j, first=(j < 2), last=False)

    def body4(g, carry):
        j0 = g * 4
        for b in range(4):
            do_chunk(j0 + b, b, first=False, last=False)
        return carry

    lax.fori_loop(1, (_NCHUNK - 5) // 4, body4, 0)
    for j in range(_NCHUNK - 5, _NCHUNK):
        do_chunk(j, j % 4, first=False, last=(j + 2 >= _NCHUNK))
    for j in range(_NCHUNK - 2, _NCHUNK):
        scatter_wait(j, j % 4)
    plsc.subcore_barrier()

    pltpu.sync_copy(acc.at[pl.ds(s * _RPT, _RPT)],
                    out_hbm.at[c, pl.ds(s * _RPT, _RPT)])


_NP2 = _NPAD // 2
_NR2 = _N // 2


def _bn_relu2(hL, hR, g, t):
    sm = jnp.sum(hL, 0, keepdims=True) + jnp.sum(hR, 0, keepdims=True)
    m = sm / _N
    s2 = (jnp.sum(hL * hL, 0, keepdims=True)
          + jnp.sum(hR * hR, 0, keepdims=True))
    v = s2 / _N - m * m
    inv = g / jnp.sqrt(v + 1e-5)
    fL = jnp.maximum((hL - m) * inv + t, 0.0)
    fR = jnp.maximum((hR - m) * inv + t, 0.0)
    return fL, fR


def _mm_body(x_ref, w_ref, o_ref):
    yL = jnp.dot(x_ref[:, :_F], w_ref[...], preferred_element_type=jnp.float32)
    yR = jnp.dot(x_ref[:, _F:], w_ref[...], preferred_element_type=jnp.float32)
    o_ref[:_NR2] = jnp.concatenate([yL, yR], axis=1)
    o_ref[_NR2:] = jnp.zeros((_NP2 - _NR2, 2 * _H), jnp.float32)


def _mlp_core(y_ref, p_ref, ba, ga, ta, Wb_ref, bb, gb, tb):
    pre = y_ref[:_NR2] + p_ref[0, :_NR2] + p_ref[1, :_NR2]
    hL, hR = _bn_relu2(pre[:, :_H] + ba, pre[:, _H:] + ba, ga, ta)
    zL = jnp.dot(hL, Wb_ref[...], preferred_element_type=jnp.float32) + bb
    zR = jnp.dot(hR, Wb_ref[...], preferred_element_type=jnp.float32) + bb
    return _bn_relu2(zL, zR, gb, tb)


def _mlp_body(y_ref, p_ref, ba_ref, ga_ref, ta_ref,
              Wb_ref, bb_ref, gb_ref, tb_ref, Wnext_ref, o_ref):
    hL, hR = _mlp_core(y_ref, p_ref, ba_ref[...], ga_ref[...], ta_ref[...],
                       Wb_ref, bb_ref[...], gb_ref[...], tb_ref[...])
    yL = jnp.dot(hL, Wnext_ref[...], preferred_element_type=jnp.float32)
    yR = jnp.dot(hR, Wnext_ref[...], preferred_element_type=jnp.float32)
    o_ref[:_NR2] = jnp.concatenate([yL, yR], axis=1)
    o_ref[_NR2:] = jnp.zeros((_NP2 - _NR2, 2 * _H), jnp.float32)


def _tail_body(y_ref, q_ref, bE_ref, bO_ref, ba_ref, ga_ref, ta_ref,
               Wb_ref, bb_ref, gb_ref, tb_ref,
               Wc1_ref, bc1_ref, gc1_ref, tc1_ref, Wc2_ref, bc2_ref, o_ref):
    hL, hR = _mlp_core(y_ref, q_ref, ba_ref[...], ga_ref[...], ta_ref[...],
                       Wb_ref, bb_ref[...], gb_ref[...], tb_ref[...])
    gids = lax.broadcasted_iota(jnp.int32, (_G, _NR2), 0)
    ohE = (gids == bE_ref[...]).astype(jnp.float32)
    ohO = (gids == bO_ref[...]).astype(jnp.float32)
    sums = (jnp.dot(ohE, hL, preferred_element_type=jnp.float32)
            + jnp.dot(ohO, hR, preferred_element_type=jnp.float32))
    cnt = (jnp.sum(ohE, axis=1, keepdims=True)
           + jnp.sum(ohO, axis=1, keepdims=True))
    hm = sums / jnp.maximum(cnt, 1.0)
    z = jnp.dot(hm, Wc1_ref[...], preferred_element_type=jnp.float32) + bc1_ref[...]
    m = jnp.mean(z, axis=0, keepdims=True)
    v = jnp.mean((z - m) * (z - m), axis=0, keepdims=True)
    z = jnp.maximum((z - m) / jnp.sqrt(v + 1e-5) * gc1_ref[...] + tc1_ref[...], 0.0)
    z = jnp.dot(z, Wc2_ref[...], preferred_element_type=jnp.float32) + bc2_ref[...]
    zmax = jnp.max(z, axis=1, keepdims=True)
    zs = z - zmax
    o_ref[...] = zs - jnp.log(jnp.sum(jnp.exp(zs), axis=1, keepdims=True))


_mm = pl.pallas_call(
    _mm_body, out_shape=jax.ShapeDtypeStruct((_NP2, 2 * _H), jnp.float32))

_mlp = pl.pallas_call(
    _mlp_body, out_shape=jax.ShapeDtypeStruct((_NP2, 2 * _H), jnp.float32))

_tail = pl.pallas_call(
    _tail_body, out_shape=jax.ShapeDtypeStruct((_G, _C), jnp.float32))


def kernel(x, edge_index, batch, W1a, b1a, g1a, t1a, W1b, b1b, g1b, t1b,
           W2a, b2a, g2a, t2a, W2b, b2b, g2b, t2b,
           Wc1, bc1, gc1, tc1, Wc2, bc2):
    r = lambda v: v.reshape(1, -1)

    ei4 = edge_index.reshape(2, _NW, _NCHUNK, _K)
    xp = x.reshape(_NR2, 2 * _F)
    bpair = batch.reshape(_NR2, 2)
    bE = bpair[:, 0].reshape(1, _NR2)
    bO = bpair[:, 1].reshape(1, _NR2)

    tosc = lambda v: v.reshape(_NPAD, _H)
    topair = lambda v: v.reshape(_NC, _NP2, 2 * _H)

    es = jax.jit(_edge_scatter)
    y1 = _mm(xp, W1a)
    p = es(tosc(y1), ei4)
    y2 = _mlp(y1, topair(p), r(b1a), r(g1a), r(t1a),
              W1b, r(b1b), r(g1b), r(t1b), W2a)
    q = es(tosc(y2), ei4)
    return _tail(y2, topair(q), bE, bO, r(b2a), r(g2a), r(t2a),
                 W2b, r(b2b), r(g2b), r(t2b),
                 Wc1, r(bc1), r(gc1), r(tc1), Wc2, r(bc2))
